# baseline (speedup 1.0000x reference)
import functools

import jax
import jax.numpy as jnp
from jax.experimental import pallas as pl
from jax.experimental.pallas import tpu as pltpu

B, T, D = 4, 2048, 4096
E, R = 8, 64
OUT3 = 4096 * 3
ALPHA = 128.0


POOL_TT = 512
N_POOL_T = T // POOL_TT


def _pool_logits_body(tok_ref, wg_ref, logits_ref):
    b = pl.program_id(0)
    t = pl.program_id(1)

    @pl.when(t == 0)
    def _init():
        logits_ref[b, :] = jnp.zeros((E,), jnp.float32)

    colsum = jnp.sum(tok_ref[0], axis=0, keepdims=True)
    partial = jnp.dot(colsum, wg_ref[...],
                      preferred_element_type=jnp.float32)
    logits_ref[b, :] += partial[0] * (1.0 / T)


def _pool_logits(tokens, Wg):
    return pl.pallas_call(
        _pool_logits_body,
        grid=(B, N_POOL_T),
        in_specs=[
            pl.BlockSpec((1, POOL_TT, D), lambda b, t: (b, t, 0)),
            pl.BlockSpec((D, E), lambda b, t: (0, 0)),
        ],
        out_specs=pl.BlockSpec((B, E), lambda b, t: (0, 0)),
        out_shape=jax.ShapeDtypeStruct((B, E), jnp.float32),
    )(tokens, Wg)


def _routing_body(logits_ref, sel_ref, ew_ref, imp_ref, load_ref, w_ref):
    logits = logits_ref[...]
    m = jnp.max(logits, axis=1, keepdims=True)
    ex = jnp.exp(logits - m)
    probs = ex / jnp.sum(ex, axis=1, keepdims=True)
    top1 = jnp.max(probs, axis=1, keepdims=True)
    eidx = jax.lax.broadcasted_iota(jnp.int32, (B, E), 1)
    is_max = probs >= top1
    sel = jnp.min(jnp.where(is_max, eidx, E), axis=1, keepdims=True)
    onehot = (eidx == sel).astype(jnp.float32)
    ew = onehot * top1
    sel_ref[...] = sel
    ew_ref[...] = ew
    imp_ref[...] = jnp.sum(ew, axis=0, keepdims=True)
    load_ref[...] = jnp.sum(onehot, axis=0, keepdims=True) * (1.0 / B)
    w_ref[...] = top1 * (ALPHA / R)


def _routing(logits):
    return pl.pallas_call(
        _routing_body,
        out_shape=(
            jax.ShapeDtypeStruct((B, 1), jnp.int32),
            jax.ShapeDtypeStruct((B, E), jnp.float32),
            jax.ShapeDtypeStruct((1, E), jnp.float32),
            jax.ShapeDtypeStruct((1, E), jnp.float32),
            jax.ShapeDtypeStruct((B, 1), jnp.float32),
        ),
    )(logits)


TT = 512
OT = 2048
N_T = T // TT
N_O = OUT3 // OT


def _lora_body(sel_ref, tok_ref, a_ref, bw_ref, w_ref, out_ref, h_ref):
    b = pl.program_id(0)
    o = pl.program_id(2)

    @pl.when(o == 0)
    def _compute_h():
        x = tok_ref[0]
        a = a_ref[0]
        h = jax.lax.dot_general(x, a, (((1,), (1,)), ((), ())),
                                preferred_element_type=jnp.float32)
        h_ref[...] = h * w_ref[b, 0]

    bw = bw_ref[0]
    out_ref[0] = jax.lax.dot_general(h_ref[...], bw,
                                     (((1,), (1,)), ((), ())),
                                     preferred_element_type=jnp.float32)


def _lora_update(tokens, A, Bw, sel, w):
    grid_spec = pltpu.PrefetchScalarGridSpec(
        num_scalar_prefetch=1,
        grid=(B, N_T, N_O),
        in_specs=[
            pl.BlockSpec((1, TT, D), lambda b, t, o, sel_ref: (b, t, 0)),
            pl.BlockSpec((1, R, D), lambda b, t, o, sel_ref: (sel_ref[b], 0, 0)),
            pl.BlockSpec((1, OT, R), lambda b, t, o, sel_ref: (sel_ref[b], o, 0)),
            pl.BlockSpec((B, 1), lambda b, t, o, sel_ref: (0, 0)),
        ],
        out_specs=pl.BlockSpec((1, TT, OT), lambda b, t, o, sel_ref: (b, t, o)),
        scratch_shapes=[pltpu.VMEM((TT, R), jnp.float32)],
    )
    return pl.pallas_call(
        _lora_body,
        grid_spec=grid_spec,
        out_shape=jax.ShapeDtypeStruct((B, T, OUT3), jnp.float32),
        compiler_params=pltpu.CompilerParams(
            dimension_semantics=("parallel", "arbitrary", "arbitrary"),
        ),
    )(sel, tokens, A, Bw, w)


def kernel(tokens, Wg, A, Bw):
    router_logits = _pool_logits(tokens, Wg)
    sel2d, expert_weights, imp2d, load2d, w = _routing(router_logits)
    sel = sel2d[:, 0]
    weighted_update = _lora_update(tokens, A, Bw, sel, w)
    return (weighted_update, router_logits, sel2d, expert_weights,
            imp2d[0], load2d[0])


# speedup vs baseline: 1.1487x; 1.1487x over previous
import functools

import jax
import jax.numpy as jnp
from jax.experimental import pallas as pl
from jax.experimental.pallas import tpu as pltpu

B, T, D = 4, 2048, 4096
E, R = 8, 64
OUT3 = 4096 * 3
ALPHA = 128.0


POOL_TT = 512
N_POOL_T = T // POOL_TT


def _pool_logits_body(tok_ref, wg_ref, logits_ref):
    b = pl.program_id(0)
    t = pl.program_id(1)

    @pl.when(t == 0)
    def _init():
        logits_ref[b, :] = jnp.zeros((E,), jnp.float32)

    colsum = jnp.sum(tok_ref[0], axis=0, keepdims=True)
    partial = jnp.dot(colsum, wg_ref[...],
                      preferred_element_type=jnp.float32)
    logits_ref[b, :] += partial[0] * (1.0 / T)


def _pool_logits(tokens, Wg):
    return pl.pallas_call(
        _pool_logits_body,
        grid=(B, N_POOL_T),
        in_specs=[
            pl.BlockSpec((1, POOL_TT, D), lambda b, t: (b, t, 0)),
            pl.BlockSpec((D, E), lambda b, t: (0, 0)),
        ],
        out_specs=pl.BlockSpec((B, E), lambda b, t: (0, 0)),
        out_shape=jax.ShapeDtypeStruct((B, E), jnp.float32),
    )(tokens, Wg)


def _routing_body(logits_ref, sel_ref, ew_ref, imp_ref, load_ref, w_ref):
    logits = logits_ref[...]
    m = jnp.max(logits, axis=1, keepdims=True)
    ex = jnp.exp(logits - m)
    probs = ex / jnp.sum(ex, axis=1, keepdims=True)
    top1 = jnp.max(probs, axis=1, keepdims=True)
    eidx = jax.lax.broadcasted_iota(jnp.int32, (B, E), 1)
    is_max = probs >= top1
    sel = jnp.min(jnp.where(is_max, eidx, E), axis=1, keepdims=True)
    onehot = (eidx == sel).astype(jnp.float32)
    ew = onehot * top1
    sel_ref[...] = sel
    ew_ref[...] = ew
    imp_ref[...] = jnp.sum(ew, axis=0, keepdims=True)
    load_ref[...] = jnp.sum(onehot, axis=0, keepdims=True) * (1.0 / B)
    w_ref[...] = top1 * (ALPHA / R)


def _routing(logits):
    return pl.pallas_call(
        _routing_body,
        out_shape=(
            jax.ShapeDtypeStruct((B, 1), jnp.int32),
            jax.ShapeDtypeStruct((B, E), jnp.float32),
            jax.ShapeDtypeStruct((1, E), jnp.float32),
            jax.ShapeDtypeStruct((1, E), jnp.float32),
            jax.ShapeDtypeStruct((B, 1), jnp.float32),
        ),
    )(logits)


HTT = 512
N_HT = T // HTT


def _h_body(sel_ref, tok_ref, a_ref, w_ref, h_ref):
    b = pl.program_id(0)
    x = tok_ref[0].astype(jnp.bfloat16)
    a = a_ref[0].astype(jnp.bfloat16)
    h = jax.lax.dot_general(x, a, (((1,), (1,)), ((), ())),
                            preferred_element_type=jnp.float32)
    h_ref[0] = (h * w_ref[b, 0]).astype(jnp.bfloat16)


def _h_pass(tokens, A, sel, w):
    grid_spec = pltpu.PrefetchScalarGridSpec(
        num_scalar_prefetch=1,
        grid=(B, N_HT),
        in_specs=[
            pl.BlockSpec((1, HTT, D), lambda b, t, sel_ref: (b, t, 0)),
            pl.BlockSpec((1, R, D), lambda b, t, sel_ref: (sel_ref[b], 0, 0)),
            pl.BlockSpec((B, 1), lambda b, t, sel_ref: (0, 0)),
        ],
        out_specs=pl.BlockSpec((1, HTT, R), lambda b, t, sel_ref: (b, t, 0)),
    )
    return pl.pallas_call(
        _h_body,
        grid_spec=grid_spec,
        out_shape=jax.ShapeDtypeStruct((B, T, R), jnp.bfloat16),
        compiler_params=pltpu.CompilerParams(
            dimension_semantics=("parallel", "parallel"),
        ),
    )(sel, tokens, A, w)


UTT = 512
OT = 2048
N_UT = T // UTT
N_O = OUT3 // OT


def _update_body(sel_ref, h_ref, bw_ref, out_ref):
    bw = bw_ref[0].astype(jnp.bfloat16)
    out_ref[0] = jax.lax.dot_general(h_ref[0], bw,
                                     (((1,), (1,)), ((), ())),
                                     preferred_element_type=jnp.float32)


def _update_pass(h, Bw, sel):
    grid_spec = pltpu.PrefetchScalarGridSpec(
        num_scalar_prefetch=1,
        grid=(B, N_O, N_UT),
        in_specs=[
            pl.BlockSpec((1, UTT, R), lambda b, o, t, sel_ref: (b, t, 0)),
            pl.BlockSpec((1, OT, R), lambda b, o, t, sel_ref: (sel_ref[b], o, 0)),
        ],
        out_specs=pl.BlockSpec((1, UTT, OT), lambda b, o, t, sel_ref: (b, t, o)),
    )
    return pl.pallas_call(
        _update_body,
        grid_spec=grid_spec,
        out_shape=jax.ShapeDtypeStruct((B, T, OUT3), jnp.float32),
        compiler_params=pltpu.CompilerParams(
            dimension_semantics=("parallel", "parallel", "arbitrary"),
        ),
    )(sel, h, Bw)


def kernel(tokens, Wg, A, Bw):
    router_logits = _pool_logits(tokens, Wg)
    sel2d, expert_weights, imp2d, load2d, w = _routing(router_logits)
    sel = sel2d[:, 0]
    h = _h_pass(tokens, A, sel, w)
    weighted_update = _update_pass(h, Bw, sel)
    return (weighted_update, router_logits, sel2d, expert_weights,
            imp2d[0], load2d[0])


# speedup vs baseline: 1.3033x; 1.1346x over previous
import functools

import jax
import jax.numpy as jnp
from jax.experimental import pallas as pl
from jax.experimental.pallas import tpu as pltpu

B, T, D = 4, 2048, 4096
E, R = 8, 64
OUT3 = 4096 * 3
ALPHA = 128.0


POOL_TT = 512
N_POOL_T = T // POOL_TT


def _pool_logits_body(tok_ref, wg_ref, logits_ref):
    b = pl.program_id(0)
    t = pl.program_id(1)

    @pl.when(t == 0)
    def _init():
        logits_ref[b, :] = jnp.zeros((E,), jnp.float32)

    colsum = jnp.sum(tok_ref[0], axis=0, keepdims=True)
    partial = jnp.dot(colsum, wg_ref[...],
                      preferred_element_type=jnp.float32)
    logits_ref[b, :] += partial[0] * (1.0 / T)


def _pool_logits(tokens, Wg):
    return pl.pallas_call(
        _pool_logits_body,
        grid=(B, N_POOL_T),
        in_specs=[
            pl.BlockSpec((1, POOL_TT, D), lambda b, t: (b, t, 0)),
            pl.BlockSpec((D, E), lambda b, t: (0, 0)),
        ],
        out_specs=pl.BlockSpec((B, E), lambda b, t: (0, 0)),
        out_shape=jax.ShapeDtypeStruct((B, E), jnp.float32),
    )(tokens, Wg)


def _routing_body(logits_ref, sel_ref, ew_ref, imp_ref, load_ref, w_ref):
    logits = logits_ref[...]
    m = jnp.max(logits, axis=1, keepdims=True)
    ex = jnp.exp(logits - m)
    probs = ex / jnp.sum(ex, axis=1, keepdims=True)
    top1 = jnp.max(probs, axis=1, keepdims=True)
    eidx = jax.lax.broadcasted_iota(jnp.int32, (B, E), 1)
    is_max = probs >= top1
    sel = jnp.min(jnp.where(is_max, eidx, E), axis=1, keepdims=True)
    onehot = (eidx == sel).astype(jnp.float32)
    ew = onehot * top1
    sel_ref[...] = sel
    ew_ref[...] = ew
    imp_ref[...] = jnp.sum(ew, axis=0, keepdims=True)
    load_ref[...] = jnp.sum(onehot, axis=0, keepdims=True) * (1.0 / B)
    w_ref[...] = top1 * (ALPHA / R)


def _routing(logits):
    return pl.pallas_call(
        _routing_body,
        out_shape=(
            jax.ShapeDtypeStruct((B, 1), jnp.int32),
            jax.ShapeDtypeStruct((B, E), jnp.float32),
            jax.ShapeDtypeStruct((1, E), jnp.float32),
            jax.ShapeDtypeStruct((1, E), jnp.float32),
            jax.ShapeDtypeStruct((B, 1), jnp.float32),
        ),
    )(logits)


UTT = 256
N_UT = T // UTT


def _lora_body(sel_ref, tok_ref, a_ref, bw_ref, w_ref, out_ref):
    b = pl.program_id(0)
    x = tok_ref[0].astype(jnp.bfloat16)
    a = a_ref[0].astype(jnp.bfloat16)
    h = jax.lax.dot_general(x, a, (((1,), (1,)), ((), ())),
                            preferred_element_type=jnp.float32)
    hb = (h * w_ref[b, 0]).astype(jnp.bfloat16)
    bw = bw_ref[0].astype(jnp.bfloat16)
    out_ref[0] = jax.lax.dot_general(hb, bw, (((1,), (1,)), ((), ())),
                                     preferred_element_type=jnp.float32)


def _lora_update(tokens, A, Bw, sel, w):
    grid_spec = pltpu.PrefetchScalarGridSpec(
        num_scalar_prefetch=1,
        grid=(B, N_UT),
        in_specs=[
            pl.BlockSpec((1, UTT, D), lambda b, t, sel_ref: (b, t, 0)),
            pl.BlockSpec((1, R, D), lambda b, t, sel_ref: (sel_ref[b], 0, 0)),
            pl.BlockSpec((1, OUT3, R), lambda b, t, sel_ref: (sel_ref[b], 0, 0)),
            pl.BlockSpec((B, 1), lambda b, t, sel_ref: (0, 0)),
        ],
        out_specs=pl.BlockSpec((1, UTT, OUT3), lambda b, t, sel_ref: (b, t, 0)),
    )
    return pl.pallas_call(
        _lora_body,
        grid_spec=grid_spec,
        out_shape=jax.ShapeDtypeStruct((B, T, OUT3), jnp.float32),
        compiler_params=pltpu.CompilerParams(
            dimension_semantics=("parallel", "arbitrary"),
        ),
    )(sel, tokens, A, Bw, w)


def kernel(tokens, Wg, A, Bw):
    router_logits = _pool_logits(tokens, Wg)
    sel2d, expert_weights, imp2d, load2d, w = _routing(router_logits)
    sel = sel2d[:, 0]
    weighted_update = _lora_update(tokens, A, Bw, sel, w)
    return (weighted_update, router_logits, sel2d, expert_weights,
            imp2d[0], load2d[0])
